# baseline (speedup 1.0000x reference)
import functools

import jax
import jax.numpy as jnp
from jax import lax
from jax.experimental import pallas as pl
from jax.experimental.pallas import tpu as pltpu
from jax.experimental.pallas import tpu_sc as plsc

PAD = 1
V = 100000
G = 4
DIV = -0.5
BSZ = 32
BEAM = 8
ROWS = BSZ * BEAM
NC, NS, L = 2, 16, 16
NW = NC * NS
RPW = ROWS // NW
HALF = V // 2
NV = HALF // L
CAPL = 48
CAP = CAPL * L
NEG = float("-inf")


CW = 1408
NCH = 71
TAIL = 32
NBUF = 2
SBV = 22
SB = (CW // L) // SBV


def _sc_body(lp_hbm, lpt_hbm, vals_hbm, idx_hbm, b0, b1, bt, cv, ci,
             s0, s1, st):
    b = lax.axis_index("s") * NC + lax.axis_index("c")
    bufs = [b0, b1]
    sems = [s0, s1]
    neg16 = jnp.full((L,), NEG, jnp.float32)
    lane = lax.iota(jnp.int32, L)
    r8 = pl.multiple_of(b * BEAM, 8)

    def start(c, q):
        col = pl.multiple_of(c * CW, 128)
        pltpu.async_copy(
            lp_hbm.at[pl.ds(r8, BEAM), pl.ds(col, CW)], bufs[q], sems[q])

    def wait(q):
        pltpu.make_async_copy(
            lp_hbm.at[pl.ds(0, BEAM), pl.ds(0, CW)], bufs[q], sems[q]).wait()

    perms = [jnp.arange(L, dtype=jnp.int32) ^ k for k in (8, 4, 2, 1)]

    def lanemin(vec):
        for p in perms:
            vec = jnp.minimum(vec, vec.at[p].get(mode="promise_in_bounds"))
        return vec

    for q in range(NBUF):
        start(q, q)
    tb = pl.multiple_of(b * BEAM * TAIL, 8)
    pltpu.async_copy(lpt_hbm.at[pl.ds(tb, BEAM * TAIL)], bt, st)

    def clr(i, _):
        for u in range(4):
            cv[pl.ds((i * 4 + u) * L, L)] = neg16
            ci[pl.ds((i * 4 + u) * L, L)] = jnp.zeros((L,), jnp.int32)
        return 0

    lax.fori_loop(0, BEAM * CAPL // 4, clr, 0)

    wait(0)
    accs = []
    for bm in range(BEAM):
        def wbody(si, cmx, bm=bm):
            vs = [b0[bm, pl.ds(si * (SBV * L) + u * L, L)]
                  for u in range(SBV)]
            while len(vs) > 1:
                nxt = [jnp.maximum(vs[k], vs[k + 1])
                       for k in range(0, len(vs) - 1, 2)]
                if len(vs) % 2:
                    nxt.append(vs[-1])
                vs = nxt
            return jnp.maximum(cmx, vs[0])
        accs.append(lax.fori_loop(0, SB, wbody, neg16))
    tvecs = [lanemin(a) for a in accs]

    def proc(buf, c, carry):
        accs, tvecs, kls = carry
        accs, tvecs, kls = list(accs), list(tvecs), list(kls)
        for bm in range(BEAM):
            acc, tv, kl = accs[bm], tvecs[bm], kls[bm]

            def sbody(si, st, bm=bm, buf=buf, c=c, tv=tv):
                acc, kl = st
                vs = []
                morr = None
                for u in range(SBV):
                    v = buf[bm, pl.ds(si * (SBV * L) + u * L, L)]
                    acc = jnp.maximum(acc, v)
                    m = v >= tv
                    morr = m if morr is None else (morr | m)
                    vs.append(v)

                def scan(kl, vs=vs, c=c, tv=tv, si=si, bm=bm):
                    for u in range(SBV):
                        v = vs[u]
                        m = v >= tv
                        iv = jnp.full((L,), c * CW + u * L, jnp.int32) \
                            + si * (SBV * L) + lane
                        pos = kl * L + lane + bm * CAP
                        plsc.store_scatter(cv, [pos], v, mask=m)
                        plsc.store_scatter(ci, [pos], iv, mask=m)
                        kl = jnp.minimum(kl + m.astype(jnp.int32), CAPL - 1)
                    return kl

                kl = lax.cond(jnp.any(morr), scan, lambda kl: kl, kl)
                return (acc, kl)

            acc, kl = lax.fori_loop(0, SB, sbody, (acc, kl))
            accs[bm] = acc
            kls[bm] = kl
            tvecs[bm] = lanemin(acc)
        return (tuple(accs), tuple(tvecs), tuple(kls))

    z16 = jnp.zeros((L,), jnp.int32)
    carry = (tuple(accs), tuple(tvecs), tuple(z16 for _ in range(BEAM)))

    def body(k, carry):
        for q in range(NBUF):
            c = k * NBUF + q
            if q != 0:
                wait(q)
            else:
                @pl.when(k > 0)
                def _():
                    wait(0)
            carry = proc(bufs[q], c, carry)
            nc = c + NBUF

            @pl.when(nc < NCH)
            def _(nc=nc, q=q):
                start(nc, q)
        return carry

    carry = lax.fori_loop(0, NCH // NBUF, body, carry)
    accs, tvecs, kls = list(carry[0]), list(carry[1]), list(carry[2])
    for q in range(NCH % NBUF):
        c = (NCH // NBUF) * NBUF + q
        wait(q)
        accs, tvecs, kls = (list(x) for x in proc(bufs[q], c, (accs, tvecs, kls)))

    pltpu.make_async_copy(
        lpt_hbm.at[pl.ds(0, BEAM * TAIL)], bt, st).wait()
    for bm in range(BEAM):
        kl = kls[bm]
        for u in range(TAIL // L):
            v = bt[pl.ds(bm * TAIL + u * L, L)]
            m = v >= tvecs[bm]
            iv = jnp.full((L,), NCH * CW + u * L, jnp.int32) + lane
            pos = kl * L + lane + bm * CAP
            plsc.store_scatter(cv, [pos], v, mask=m)
            plsc.store_scatter(ci, [pos], iv, mask=m)
            kl = jnp.minimum(kl + m.astype(jnp.int32), CAPL - 1)

    ob = pl.multiple_of(b * BEAM * CAP, 8)
    pltpu.sync_copy(cv, vals_hbm.at[pl.ds(ob, BEAM * CAP)])
    pltpu.sync_copy(ci, idx_hbm.at[pl.ds(ob, BEAM * CAP)])


def _sc_candidates(lp2, lpt):
    mesh = plsc.VectorSubcoreMesh(
        core_axis_name="c", subcore_axis_name="s", num_cores=NC, num_subcores=NS
    )
    k = pl.kernel(
        _sc_body,
        out_type=[
            jax.ShapeDtypeStruct((ROWS * CAP,), jnp.float32),
            jax.ShapeDtypeStruct((ROWS * CAP,), jnp.int32),
        ],
        mesh=mesh,
        scratch_types=[
            pltpu.VMEM((BEAM, CW), jnp.float32),
            pltpu.VMEM((BEAM, CW), jnp.float32),
            pltpu.VMEM((BEAM * TAIL,), jnp.float32),
            pltpu.VMEM((BEAM * CAP,), jnp.float32),
            pltpu.VMEM((BEAM * CAP,), jnp.int32),
            pltpu.SemaphoreType.DMA,
            pltpu.SemaphoreType.DMA,
            pltpu.SemaphoreType.DMA,
        ],
        compiler_params=pltpu.CompilerParams(needs_layout_passes=False),
    )
    return k(lp2, lpt)


def _fix_body(cvals_ref, cidx_ref, bias_ref, mask_ref, so_ref, io_ref, bo_ref):
    BIG = jnp.int32(1 << 30)
    pen_toks = []
    scols = [None] * BEAM
    icols = [None] * BEAM
    bcols = [None] * BEAM
    for g in range(G):
        v0 = cvals_ref[:, g * CAP:(g + 1) * CAP] + bias_ref[:, g:g + 1]
        v1 = cvals_ref[:, (g + 4) * CAP:(g + 5) * CAP] + bias_ref[:, g + 4:g + 5]
        i0 = cidx_ref[:, g * CAP:(g + 1) * CAP]
        i1 = cidx_ref[:, (g + 4) * CAP:(g + 5) * CAP]
        if g > 0:
            p0 = jnp.zeros_like(v0)
            p1 = jnp.zeros_like(v1)
            for tk in pen_toks:
                p0 += (i0 == tk).astype(jnp.float32)
                p1 += (i1 == tk).astype(jnp.float32)
            v0 = v0 + DIV * p0
            v1 = v1 + DIV * p1
        v = jnp.concatenate([v0, v1], axis=1)
        f = jnp.concatenate([i0, i1 + V], axis=1)
        for k in range(2):
            mx = jnp.max(v, axis=1, keepdims=True)
            fi = jnp.min(jnp.where(v == mx, f, BIG), axis=1, keepdims=True)
            bm = (fi >= V).astype(jnp.int32)
            tok = fi - bm * V
            msk = jnp.where(bm == 0, mask_ref[:, g:g + 1], mask_ref[:, g + 4:g + 5])
            tokm = jnp.where(msk == 0, PAD, tok)
            scols[k * 4 + g] = mx
            icols[k * 4 + g] = tokm
            bcols[k * 4 + g] = bm * G + g
            pen_toks.append(tokm)
            if k == 0:
                v = jnp.where(f == fi, NEG, v)
    so_ref[...] = jnp.concatenate(scols, axis=1)
    io_ref[...] = jnp.concatenate(icols, axis=1)
    bo_ref[...] = jnp.concatenate(bcols, axis=1)


def _fixup(cvals, cidx, bias, mask, interpret=False):
    return pl.pallas_call(
        _fix_body,
        out_shape=[
            jax.ShapeDtypeStruct((BSZ, BEAM), jnp.float32),
            jax.ShapeDtypeStruct((BSZ, BEAM), jnp.int32),
            jax.ShapeDtypeStruct((BSZ, BEAM), jnp.int32),
        ],
        interpret=interpret,
    )(cvals, cidx, bias, mask)


def kernel(step, lprobs, mask_stop_search, scores, prev_indices, original_batch_idxs):
    lp2 = lprobs.reshape(ROWS, V)
    lpt = lp2[:, NCH * CW:].reshape(ROWS * TAIL)
    cand_vals, cand_idx = _sc_candidates(lp2, lpt)
    step_i = jnp.asarray(step, jnp.int32)
    sc_step = lax.dynamic_index_in_dim(scores, step_i, axis=2, keepdims=False)
    m0 = (jnp.arange(BEAM, dtype=jnp.int32) // 4) == 0
    bias = jnp.where(step_i == 0, jnp.where(m0[None, :], 0.0, NEG), sc_step)
    out = _fixup(
        cand_vals.reshape(BSZ, BEAM * CAP),
        cand_idx.reshape(BSZ, BEAM * CAP),
        bias.astype(jnp.float32),
        mask_stop_search,
    )
    return (out[0], out[1], out[2])


# speedup vs baseline: 1.2062x; 1.2062x over previous
import functools

import jax
import jax.numpy as jnp
from jax import lax
from jax.experimental import pallas as pl
from jax.experimental.pallas import tpu as pltpu
from jax.experimental.pallas import tpu_sc as plsc

PAD = 1
V = 100000
G = 4
DIV = -0.5
BSZ = 32
BEAM = 8
ROWS = BSZ * BEAM
NC, NS, L = 2, 16, 16
NW = NC * NS
RPW = ROWS // NW
HALF = V // 2
NV = HALF // L
CAPL = 48
CAP = CAPL * L
NEG = float("-inf")


CW = 1408
NCH = 71
TAIL = 32
NBUF = 2
SBV = 11
SB = (CW // L) // SBV


def _sc_body(lp_hbm, lpt_hbm, vals_hbm, idx_hbm, b0, b1, bt, cv, ci,
             s0, s1, st):
    b = lax.axis_index("s") * NC + lax.axis_index("c")
    bufs = [b0, b1]
    sems = [s0, s1]
    neg16 = jnp.full((L,), NEG, jnp.float32)
    lane = lax.iota(jnp.int32, L)
    r8 = pl.multiple_of(b * BEAM, 8)

    def start(c, q):
        col = pl.multiple_of(c * CW, 128)
        pltpu.async_copy(
            lp_hbm.at[pl.ds(r8, BEAM), pl.ds(col, CW)], bufs[q], sems[q])

    def wait(q):
        pltpu.make_async_copy(
            lp_hbm.at[pl.ds(0, BEAM), pl.ds(0, CW)], bufs[q], sems[q]).wait()

    perms = [jnp.arange(L, dtype=jnp.int32) ^ k for k in (8, 4, 2, 1)]

    def lanemin(vec):
        for p in perms:
            vec = jnp.minimum(vec, vec.at[p].get(mode="promise_in_bounds"))
        return vec

    for q in range(NBUF):
        start(q, q)
    tb = pl.multiple_of(b * BEAM * TAIL, 8)
    pltpu.async_copy(lpt_hbm.at[pl.ds(tb, BEAM * TAIL)], bt, st)

    def clr(i, _):
        for u in range(4):
            cv[pl.ds((i * 4 + u) * L, L)] = neg16
            ci[pl.ds((i * 4 + u) * L, L)] = jnp.zeros((L,), jnp.int32)
        return 0

    lax.fori_loop(0, BEAM * CAPL // 4, clr, 0)

    wait(0)
    accs = []
    for bm in range(BEAM):
        def wbody(si, cmx, bm=bm):
            vs = [b0[bm, pl.ds(si * (SBV * L) + u * L, L)]
                  for u in range(SBV)]
            while len(vs) > 1:
                nxt = [jnp.maximum(vs[k], vs[k + 1])
                       for k in range(0, len(vs) - 1, 2)]
                if len(vs) % 2:
                    nxt.append(vs[-1])
                vs = nxt
            return jnp.maximum(cmx, vs[0])
        accs.append(lax.fori_loop(0, SB, wbody, neg16))
    tvecs = [lanemin(a) for a in accs]

    def proc(buf, c, carry):
        accs, tvecs, kls = carry
        accs, tvecs, kls = list(accs), list(tvecs), list(kls)
        for bm in range(BEAM):
            acc, tv, kl = accs[bm], tvecs[bm], kls[bm]

            def sbody(si, st, bm=bm, buf=buf, c=c, tv=tv):
                acc, kl = st
                vs = []
                morr = None
                for u in range(SBV):
                    v = buf[bm, pl.ds(si * (SBV * L) + u * L, L)]
                    acc = jnp.maximum(acc, v)
                    m = v >= tv
                    morr = m if morr is None else (morr | m)
                    vs.append(v)

                def scan(kl, vs=vs, c=c, tv=tv, si=si, bm=bm):
                    for u in range(SBV):
                        v = vs[u]
                        m = v >= tv
                        iv = jnp.full((L,), c * CW + u * L, jnp.int32) \
                            + si * (SBV * L) + lane
                        pos = kl * L + lane + bm * CAP
                        plsc.store_scatter(cv, [pos], v, mask=m)
                        plsc.store_scatter(ci, [pos], iv, mask=m)
                        kl = jnp.minimum(kl + m.astype(jnp.int32), CAPL - 1)
                    return kl

                kl = lax.cond(jnp.any(morr), scan, lambda kl: kl, kl)
                return (acc, kl)

            acc, kl = lax.fori_loop(0, SB, sbody, (acc, kl))
            accs[bm] = acc
            kls[bm] = kl
            tvecs[bm] = lanemin(acc)
        return (tuple(accs), tuple(tvecs), tuple(kls))

    z16 = jnp.zeros((L,), jnp.int32)
    carry = (tuple(accs), tuple(tvecs), tuple(z16 for _ in range(BEAM)))

    def body(k, carry):
        for q in range(NBUF):
            c = k * NBUF + q
            if q != 0:
                wait(q)
            else:
                @pl.when(k > 0)
                def _():
                    wait(0)
            carry = proc(bufs[q], c, carry)
            nc = c + NBUF

            @pl.when(nc < NCH)
            def _(nc=nc, q=q):
                start(nc, q)
        return carry

    carry = lax.fori_loop(0, NCH // NBUF, body, carry)
    accs, tvecs, kls = list(carry[0]), list(carry[1]), list(carry[2])
    for q in range(NCH % NBUF):
        c = (NCH // NBUF) * NBUF + q
        wait(q)
        accs, tvecs, kls = (list(x) for x in proc(bufs[q], c, (accs, tvecs, kls)))

    pltpu.make_async_copy(
        lpt_hbm.at[pl.ds(0, BEAM * TAIL)], bt, st).wait()
    for bm in range(BEAM):
        kl = kls[bm]
        for u in range(TAIL // L):
            v = bt[pl.ds(bm * TAIL + u * L, L)]
            m = v >= tvecs[bm]
            iv = jnp.full((L,), NCH * CW + u * L, jnp.int32) + lane
            pos = kl * L + lane + bm * CAP
            plsc.store_scatter(cv, [pos], v, mask=m)
            plsc.store_scatter(ci, [pos], iv, mask=m)
            kl = jnp.minimum(kl + m.astype(jnp.int32), CAPL - 1)

    ob = pl.multiple_of(b * BEAM * CAP, 8)
    pltpu.sync_copy(cv, vals_hbm.at[pl.ds(ob, BEAM * CAP)])
    pltpu.sync_copy(ci, idx_hbm.at[pl.ds(ob, BEAM * CAP)])


def _sc_candidates(lp2, lpt):
    mesh = plsc.VectorSubcoreMesh(
        core_axis_name="c", subcore_axis_name="s", num_cores=NC, num_subcores=NS
    )
    k = pl.kernel(
        _sc_body,
        out_type=[
            jax.ShapeDtypeStruct((ROWS * CAP,), jnp.float32),
            jax.ShapeDtypeStruct((ROWS * CAP,), jnp.int32),
        ],
        mesh=mesh,
        scratch_types=[
            pltpu.VMEM((BEAM, CW), jnp.float32),
            pltpu.VMEM((BEAM, CW), jnp.float32),
            pltpu.VMEM((BEAM * TAIL,), jnp.float32),
            pltpu.VMEM((BEAM * CAP,), jnp.float32),
            pltpu.VMEM((BEAM * CAP,), jnp.int32),
            pltpu.SemaphoreType.DMA,
            pltpu.SemaphoreType.DMA,
            pltpu.SemaphoreType.DMA,
        ],
        compiler_params=pltpu.CompilerParams(needs_layout_passes=False),
    )
    return k(lp2, lpt)


def _fix_body(cvals_ref, cidx_ref, bias_ref, mask_ref, so_ref, io_ref, bo_ref):
    BIG = jnp.int32(1 << 30)
    pen_toks = []
    scols = [None] * BEAM
    icols = [None] * BEAM
    bcols = [None] * BEAM
    for g in range(G):
        v0 = cvals_ref[:, g * CAP:(g + 1) * CAP] + bias_ref[:, g:g + 1]
        v1 = cvals_ref[:, (g + 4) * CAP:(g + 5) * CAP] + bias_ref[:, g + 4:g + 5]
        i0 = cidx_ref[:, g * CAP:(g + 1) * CAP]
        i1 = cidx_ref[:, (g + 4) * CAP:(g + 5) * CAP]
        if g > 0:
            p0 = jnp.zeros_like(v0)
            p1 = jnp.zeros_like(v1)
            for tk in pen_toks:
                p0 += (i0 == tk).astype(jnp.float32)
                p1 += (i1 == tk).astype(jnp.float32)
            v0 = v0 + DIV * p0
            v1 = v1 + DIV * p1
        v = jnp.concatenate([v0, v1], axis=1)
        f = jnp.concatenate([i0, i1 + V], axis=1)
        for k in range(2):
            mx = jnp.max(v, axis=1, keepdims=True)
            fi = jnp.min(jnp.where(v == mx, f, BIG), axis=1, keepdims=True)
            bm = (fi >= V).astype(jnp.int32)
            tok = fi - bm * V
            msk = jnp.where(bm == 0, mask_ref[:, g:g + 1], mask_ref[:, g + 4:g + 5])
            tokm = jnp.where(msk == 0, PAD, tok)
            scols[k * 4 + g] = mx
            icols[k * 4 + g] = tokm
            bcols[k * 4 + g] = bm * G + g
            pen_toks.append(tokm)
            if k == 0:
                v = jnp.where(f == fi, NEG, v)
    so_ref[...] = jnp.concatenate(scols, axis=1)
    io_ref[...] = jnp.concatenate(icols, axis=1)
    bo_ref[...] = jnp.concatenate(bcols, axis=1)


def _fixup(cvals, cidx, bias, mask, interpret=False):
    return pl.pallas_call(
        _fix_body,
        out_shape=[
            jax.ShapeDtypeStruct((BSZ, BEAM), jnp.float32),
            jax.ShapeDtypeStruct((BSZ, BEAM), jnp.int32),
            jax.ShapeDtypeStruct((BSZ, BEAM), jnp.int32),
        ],
        interpret=interpret,
    )(cvals, cidx, bias, mask)


def kernel(step, lprobs, mask_stop_search, scores, prev_indices, original_batch_idxs):
    lp2 = lprobs.reshape(ROWS, V)
    lpt = lp2[:, NCH * CW:].reshape(ROWS * TAIL)
    cand_vals, cand_idx = _sc_candidates(lp2, lpt)
    step_i = jnp.asarray(step, jnp.int32)
    sc_step = lax.dynamic_index_in_dim(scores, step_i, axis=2, keepdims=False)
    m0 = (jnp.arange(BEAM, dtype=jnp.int32) // 4) == 0
    bias = jnp.where(step_i == 0, jnp.where(m0[None, :], 0.0, NEG), sc_step)
    out = _fixup(
        cand_vals.reshape(BSZ, BEAM * CAP),
        cand_idx.reshape(BSZ, BEAM * CAP),
        bias.astype(jnp.float32),
        mask_stop_search,
    )
    return (out[0], out[1], out[2])
